# baseline (speedup 1.0000x reference)
import functools

import jax
import jax.numpy as jnp
from jax import lax
from jax.experimental import pallas as pl
from jax.experimental.pallas import tpu as pltpu
from jax.experimental.pallas import tpu_sc as plsc

D = 512
NC = 2
NS = 16
NW = NC * NS
L = 16


@functools.cache
def _make_sc_lookup(B, T):
    N = B * T
    TW = T // NW
    HT = TW // 2
    BP = B // 2
    C = 2 * HT
    NCH = 2 * BP
    NB = 3
    PD = 1

    mesh = plsc.VectorSubcoreMesh(core_axis_name="c", subcore_axis_name="s",
                                  num_cores=NC, num_subcores=NS)

    @functools.partial(
        pl.kernel,
        out_type=jax.ShapeDtypeStruct((N, D), jnp.float32),
        mesh=mesh,
        scratch_types=[
            pltpu.VMEM((B * TW,), jnp.int32),
            pltpu.VMEM((HT, D), jnp.float32),
            [pltpu.VMEM((C, D), jnp.float32) for _ in range(NB)],
            pltpu.SemaphoreType.DMA,
            pltpu.SemaphoreType.DMA,
            [pltpu.SemaphoreType.DMA for _ in range(NB)],
            [pltpu.SemaphoreType.DMA for _ in range(NB)],
        ],
    )
    def lookup(ids_hbm, tok_hbm, pos_hbm, out_hbm, idx_v, pos_v, rows_v,
               psem, isem, gsem, wsem):
        wid = lax.axis_index("s") * NC + lax.axis_index("c")
        t0 = wid * TW
        pltpu.async_copy(pos_hbm.at[pl.ds(t0, HT)], pos_v, psem)
        for th in range(2):
            for b in range(B):
                pltpu.async_copy(
                    ids_hbm.at[b, pl.ds(t0 + th * HT, HT)],
                    idx_v.at[pl.ds((th * B + b) * HT, HT)], isem)
        for _ in range(2 * B):
            pltpu.make_async_copy(ids_hbm.at[0, pl.ds(0, HT)],
                                  idx_v.at[pl.ds(0, HT)], isem).wait()

        def prefetch(ch, buf):
            pltpu.async_copy(tok_hbm.at[idx_v.at[pl.ds(ch * C, C)]],
                             rows_v[buf], gsem[buf])

        for g in range(min(PD, NCH)):
            prefetch(g, g % NB)
        pltpu.make_async_copy(pos_hbm.at[pl.ds(0, HT)], pos_v, psem).wait()
        for ch in range(NCH):
            buf = ch % NB
            th, pair = divmod(ch, BP)
            g = ch + PD
            if g < NCH:
                gb = g % NB
                if g >= NB:
                    pltpu.make_async_copy(rows_v[gb],
                                          out_hbm.at[pl.ds(0, C)],
                                          wsem[gb]).wait()
                prefetch(g, gb)
            pltpu.make_async_copy(tok_hbm.at[pl.ds(0, C)], rows_v[buf],
                                  gsem[buf]).wait()
            if th == 1 and pair == 0:
                pltpu.make_async_copy(pos_hbm.at[pl.ds(0, HT)], pos_v,
                                      psem).wait()

            def row_add(r, carry, buf=buf):
                for j in range(D // L):
                    sl = pl.ds(j * L, L)
                    p = pos_v[r, sl]
                    a = rows_v[buf][r, sl]
                    b2 = rows_v[buf][HT + r, sl]
                    rows_v[buf][r, sl] = a + p
                    rows_v[buf][HT + r, sl] = b2 + p
                return carry

            lax.fori_loop(0, HT, row_add, 0)
            if th == 0 and pair == BP - 1:
                pltpu.async_copy(pos_hbm.at[pl.ds(t0 + HT, HT)], pos_v, psem)
            for i in range(2):
                b = 2 * pair + i
                pltpu.async_copy(
                    rows_v[buf].at[pl.ds(i * HT, HT)],
                    out_hbm.at[pl.ds(b * T + t0 + th * HT, HT)], wsem[buf])
        for ch in range(max(0, NCH - (NB - PD) - PD), NCH):
            buf = ch % NB
            pltpu.make_async_copy(rows_v[buf], out_hbm.at[pl.ds(0, C)],
                                  wsem[buf]).wait()

    return lookup


def kernel(input_ids, token_table, pos_table):
    B, T = input_ids.shape
    ids = input_ids.astype(jnp.int32)
    out = _make_sc_lookup(B, T)(ids, token_table, pos_table)
    return out.reshape(B, T, D)


# speedup vs baseline: 1.0554x; 1.0554x over previous
import functools

import jax
import jax.numpy as jnp
from jax import lax
from jax.experimental import pallas as pl
from jax.experimental.pallas import tpu as pltpu
from jax.experimental.pallas import tpu_sc as plsc

D = 512
NC = 2
NS = 16
NW = NC * NS
L = 16


@functools.cache
def _make_sc_lookup(B, T):
    N = B * T
    TW = T // NW
    NPW = B * TW
    C = TW
    NCH = B
    NB = 2

    mesh = plsc.VectorSubcoreMesh(core_axis_name="c", subcore_axis_name="s",
                                  num_cores=NC, num_subcores=NS)

    @functools.partial(
        pl.kernel,
        out_type=jax.ShapeDtypeStruct((N, D), jnp.float32),
        mesh=mesh,
        scratch_types=[
            pltpu.VMEM((NPW,), jnp.int32),
            pltpu.VMEM((TW, D), jnp.float32),
            [pltpu.VMEM((C, D), jnp.float32) for _ in range(NB)],
            pltpu.SemaphoreType.DMA,
            pltpu.SemaphoreType.DMA,
            [pltpu.SemaphoreType.DMA for _ in range(NB)],
            [pltpu.SemaphoreType.DMA for _ in range(NB)],
        ],
    )
    def lookup(ids_hbm, tok_hbm, pos_hbm, out_hbm, idx_v, pos_v, rows_v,
               psem, isem, gsem, wsem):
        wid = lax.axis_index("s") * NC + lax.axis_index("c")
        t0 = wid * TW
        pltpu.async_copy(pos_hbm.at[pl.ds(t0, TW)], pos_v, psem)
        for b in range(B):
            pltpu.async_copy(ids_hbm.at[b, pl.ds(t0, TW)],
                             idx_v.at[pl.ds(b * TW, TW)], isem)
        for b in range(B):
            pltpu.make_async_copy(ids_hbm.at[0, pl.ds(0, TW)],
                                  idx_v.at[pl.ds(0, TW)], isem).wait()

        def prefetch(ch, buf):
            pltpu.async_copy(tok_hbm.at[idx_v.at[pl.ds(ch * C, C)]],
                             rows_v[buf], gsem[buf])

        for g in range(min(NB - 1, NCH)):
            prefetch(g, g % NB)
        pltpu.make_async_copy(pos_hbm.at[pl.ds(0, TW)], pos_v, psem).wait()
        for ch in range(NCH):
            buf = ch % NB
            g = ch + NB - 1
            if g < NCH:
                gb = g % NB
                if g >= NB:
                    pltpu.make_async_copy(rows_v[gb],
                                          out_hbm.at[pl.ds(0, C)],
                                          wsem[gb]).wait()
                prefetch(g, gb)
            pltpu.make_async_copy(tok_hbm.at[pl.ds(0, C)], rows_v[buf],
                                  gsem[buf]).wait()

            def row_add(r, carry, buf=buf):
                for j in range(D // L):
                    plsc.addupdate(rows_v[buf].at[r, pl.ds(j * L, L)],
                                   pos_v[r, pl.ds(j * L, L)])
                return carry

            lax.fori_loop(0, C, row_add, 0)
            pltpu.async_copy(rows_v[buf], out_hbm.at[pl.ds(ch * T + t0, C)],
                             wsem[buf])
        for ch in range(max(0, NCH - NB), NCH):
            buf = ch % NB
            pltpu.make_async_copy(rows_v[buf], out_hbm.at[pl.ds(0, C)],
                                  wsem[buf]).wait()

    return lookup


def kernel(input_ids, token_table, pos_table):
    B, T = input_ids.shape
    ids = input_ids.astype(jnp.int32)
    out = _make_sc_lookup(B, T)(ids, token_table, pos_table)
    return out.reshape(B, T, D)
